# baseline (speedup 1.0000x reference)
import jax
import jax.numpy as jnp
from jax import lax
from jax.experimental import pallas as pl
from jax.experimental.pallas import tpu as pltpu

V = 100000
D = 64
H = 128
L = 200

NV = 25
R = V // NV

GB = 8
NG = L // GB


def _gather_mlp_body(idx_sref, *refs):
    emb_refs = refs[:GB]
    w1_ref, b1_ref, proj_ref, h_ref, acc_ref = refs[GB:]
    i = pl.program_id(0)

    @pl.when(i == 0)
    def _():
        acc_ref[...] = jnp.zeros_like(acc_ref)

    s = None
    for k in range(GB):
        r = idx_sref[GB * i + k] % 8
        row = emb_refs[k][pl.ds(r, 1), :]
        s = row if s is None else s + row
    acc_ref[...] += s

    @pl.when(i == NG - 1)
    def _():
        pre = lax.dot_general(acc_ref[...], w1_ref[...],
                              (((1,), (1,)), ((), ())),
                              preferred_element_type=jnp.float32)
        h = jnp.maximum(pre + b1_ref[...], 0.0)
        proj_ref[...] = h
        h_ref[...] = h


def _emb_spec(k):
    return pl.BlockSpec((8, D), lambda i, s, k=k: (s[GB * i + k] // 8, 0))


_gather_mlp = pl.pallas_call(
    _gather_mlp_body,
    grid_spec=pltpu.PrefetchScalarGridSpec(
        num_scalar_prefetch=1,
        grid=(NG,),
        in_specs=(
            [_emb_spec(k) for k in range(GB)]
            + [pl.BlockSpec((H, D), lambda i, s: (0, 0)),
               pl.BlockSpec((1, H), lambda i, s: (0, 0))]
        ),
        out_specs=[
            pl.BlockSpec((1, H), lambda i, s: (0, 0)),
            pl.BlockSpec((1, H), lambda i, s: (0, 0)),
        ],
        scratch_shapes=[pltpu.VMEM((1, D), jnp.float32)],
    ),
    out_shape=[
        jax.ShapeDtypeStruct((1, H), jnp.float32),
        jax.ShapeDtypeStruct((1, H), jnp.float32),
    ],
)


def _project_lsm_body(h_ref, w2_ref, b2_ref, out_ref):
    i = pl.program_id(0)
    h = h_ref[...].astype(jnp.bfloat16)
    w2b = w2_ref[0].astype(jnp.bfloat16)
    lg = lax.dot_general(h, w2b, (((1,), (1,)), ((), ())),
                         preferred_element_type=jnp.float32) + b2_ref[0]
    out_ref[pl.ds(i, 1), :] = lg

    @pl.when(i == NV - 1)
    def _():
        allv = out_ref[...]
        m = jnp.max(allv)
        lse = m + jnp.log(jnp.sum(jnp.exp(allv - m)))
        out_ref[...] = allv - lse


_project_lsm = pl.pallas_call(
    _project_lsm_body,
    grid=(NV,),
    in_specs=[
        pl.BlockSpec((1, H), lambda i: (0, 0)),
        pl.BlockSpec((1, R, H), lambda i: (i, 0, 0)),
        pl.BlockSpec((1, 1, R), lambda i: (i, 0, 0)),
    ],
    out_specs=pl.BlockSpec((NV, R), lambda i: (0, 0)),
    out_shape=jax.ShapeDtypeStruct((NV, R), jnp.float32),
)


def kernel(_inputs, emb, W1, b1, W2, b2):
    idx = _inputs.astype(jnp.int32)
    embs = (emb,) * GB
    proj, h = _gather_mlp(idx, *embs, W1, b1.reshape(1, H))
    outr = _project_lsm(h, W2.reshape(NV, R, H), b2.reshape(NV, 1, R))
    return (proj, outr.reshape(1, V))


# speedup vs baseline: 1.0608x; 1.0608x over previous
import jax
import jax.numpy as jnp
from jax import lax
from jax.experimental import pallas as pl
from jax.experimental.pallas import tpu as pltpu

V = 100000
D = 64
H = 128
L = 200

NV = 25
R = V // NV

GB = 50
NG = L // GB


def _gather_mlp_body(idx_sref, *refs):
    emb_refs = refs[:GB]
    w1_ref, b1_ref, proj_ref, h_ref, acc_ref = refs[GB:]
    i = pl.program_id(0)

    @pl.when(i == 0)
    def _():
        acc_ref[...] = jnp.zeros_like(acc_ref)

    s = None
    for k in range(GB):
        r = idx_sref[GB * i + k] % 8
        row = emb_refs[k][pl.ds(r, 1), :]
        s = row if s is None else s + row
    acc_ref[...] += s

    @pl.when(i == NG - 1)
    def _():
        pre = lax.dot_general(acc_ref[...], w1_ref[...],
                              (((1,), (1,)), ((), ())),
                              preferred_element_type=jnp.float32)
        h = jnp.maximum(pre + b1_ref[...], 0.0)
        proj_ref[...] = h
        h_ref[...] = h


def _emb_spec(k):
    return pl.BlockSpec((8, D), lambda i, s, k=k: (s[GB * i + k] // 8, 0))


_gather_mlp = pl.pallas_call(
    _gather_mlp_body,
    grid_spec=pltpu.PrefetchScalarGridSpec(
        num_scalar_prefetch=1,
        grid=(NG,),
        in_specs=(
            [_emb_spec(k) for k in range(GB)]
            + [pl.BlockSpec((H, D), lambda i, s: (0, 0)),
               pl.BlockSpec((1, H), lambda i, s: (0, 0))]
        ),
        out_specs=[
            pl.BlockSpec((1, H), lambda i, s: (0, 0)),
            pl.BlockSpec((1, H), lambda i, s: (0, 0)),
        ],
        scratch_shapes=[pltpu.VMEM((1, D), jnp.float32)],
    ),
    out_shape=[
        jax.ShapeDtypeStruct((1, H), jnp.float32),
        jax.ShapeDtypeStruct((1, H), jnp.float32),
    ],
)


def _project_lsm_body(h_ref, w2_ref, b2_ref, out_ref):
    i = pl.program_id(0)
    h = h_ref[...].astype(jnp.bfloat16)
    w2b = w2_ref[0].astype(jnp.bfloat16)
    lg = lax.dot_general(h, w2b, (((1,), (1,)), ((), ())),
                         preferred_element_type=jnp.float32) + b2_ref[0]
    out_ref[pl.ds(i, 1), :] = lg

    @pl.when(i == NV - 1)
    def _():
        allv = out_ref[...]
        m = jnp.max(allv)
        lse = m + jnp.log(jnp.sum(jnp.exp(allv - m)))
        out_ref[...] = allv - lse


_project_lsm = pl.pallas_call(
    _project_lsm_body,
    grid=(NV,),
    in_specs=[
        pl.BlockSpec((1, H), lambda i: (0, 0)),
        pl.BlockSpec((1, R, H), lambda i: (i, 0, 0)),
        pl.BlockSpec((1, 1, R), lambda i: (i, 0, 0)),
    ],
    out_specs=pl.BlockSpec((NV, R), lambda i: (0, 0)),
    out_shape=jax.ShapeDtypeStruct((NV, R), jnp.float32),
)


def kernel(_inputs, emb, W1, b1, W2, b2):
    idx = _inputs.astype(jnp.int32)
    embs = (emb,) * GB
    proj, h = _gather_mlp(idx, *embs, W1, b1.reshape(1, H))
    outr = _project_lsm(h, W2.reshape(NV, R, H), b2.reshape(NV, 1, R))
    return (proj, outr.reshape(1, V))


# speedup vs baseline: 1.2924x; 1.2184x over previous
import jax
import jax.numpy as jnp
from jax import lax
from jax.experimental import pallas as pl
from jax.experimental.pallas import tpu as pltpu

V = 100000
D = 64
H = 128
L = 200

NV = 10
R = V // NV
NSEM = 8


def _body(idx_sref, emb_ref, w1_ref, b1_ref, w2_ref, b2_ref,
          proj_ref, out_ref, h_ref, rows_ref, sem):
    i = pl.program_id(0)

    @pl.when(i == 0)
    def _():
        for j in range(L):
            pltpu.make_async_copy(emb_ref.at[idx_sref[j]], rows_ref.at[j],
                                  sem.at[j % NSEM]).start()
        for j in range(L):
            pltpu.make_async_copy(emb_ref.at[idx_sref[j]], rows_ref.at[j],
                                  sem.at[j % NSEM]).wait()

        e = jnp.sum(rows_ref[...], axis=0, keepdims=True)
        pre = lax.dot_general(e, w1_ref[...], (((1,), (1,)), ((), ())),
                              preferred_element_type=jnp.float32)
        h = jnp.maximum(pre + b1_ref[...], 0.0)
        h_ref[...] = h
        proj_ref[...] = h

    h = h_ref[...].astype(jnp.bfloat16)
    w2b = w2_ref[0].astype(jnp.bfloat16)
    lg = lax.dot_general(h, w2b, (((1,), (1,)), ((), ())),
                         preferred_element_type=jnp.float32) + b2_ref[0]
    out_ref[pl.ds(i, 1), :] = lg

    @pl.when(i == NV - 1)
    def _():
        allv = out_ref[...]
        m = jnp.max(allv)
        lse = m + jnp.log(jnp.sum(jnp.exp(allv - m)))
        out_ref[...] = allv - lse


_tc_fused = pl.pallas_call(
    _body,
    grid_spec=pltpu.PrefetchScalarGridSpec(
        num_scalar_prefetch=1,
        grid=(NV,),
        in_specs=[
            pl.BlockSpec(memory_space=pltpu.MemorySpace.HBM),
            pl.BlockSpec((H, D), lambda i, s: (0, 0)),
            pl.BlockSpec((1, H), lambda i, s: (0, 0)),
            pl.BlockSpec((1, R, H), lambda i, s: (i, 0, 0)),
            pl.BlockSpec((1, 1, R), lambda i, s: (i, 0, 0)),
        ],
        out_specs=[
            pl.BlockSpec((1, H), lambda i, s: (0, 0)),
            pl.BlockSpec((NV, R), lambda i, s: (0, 0)),
        ],
        scratch_shapes=[
            pltpu.VMEM((1, H), jnp.float32),
            pltpu.VMEM((L, D), jnp.float32),
            pltpu.SemaphoreType.DMA((NSEM,)),
        ],
    ),
    out_shape=[
        jax.ShapeDtypeStruct((1, H), jnp.float32),
        jax.ShapeDtypeStruct((NV, R), jnp.float32),
    ],
)


def kernel(_inputs, emb, W1, b1, W2, b2):
    idx = _inputs.astype(jnp.int32)
    proj, outr = _tc_fused(idx, emb, W1, b1.reshape(1, H),
                           W2.reshape(NV, R, H), b2.reshape(NV, 1, R))
    return (proj, outr.reshape(1, V))


# speedup vs baseline: 1.3043x; 1.0092x over previous
import jax
import jax.numpy as jnp
from jax import lax
from jax.experimental import pallas as pl
from jax.experimental.pallas import tpu as pltpu

V = 100000
D = 64
H = 128
L = 200

NV = 5
R = V // NV
NSEM = 8


def _body(idx_sref, emb_ref, w1_ref, b1_ref, w2_ref, b2_ref,
          proj_ref, out_ref, h_ref, rows_ref, sem):
    i = pl.program_id(0)

    @pl.when(i == 0)
    def _():
        for j in range(L):
            pltpu.make_async_copy(emb_ref.at[idx_sref[j]], rows_ref.at[j],
                                  sem.at[j % NSEM]).start()
        for j in range(L):
            pltpu.make_async_copy(emb_ref.at[idx_sref[j]], rows_ref.at[j],
                                  sem.at[j % NSEM]).wait()

        e = jnp.sum(rows_ref[...], axis=0, keepdims=True)
        pre = lax.dot_general(e, w1_ref[...], (((1,), (1,)), ((), ())),
                              preferred_element_type=jnp.float32)
        h = jnp.maximum(pre + b1_ref[...], 0.0)
        h_ref[...] = h
        proj_ref[...] = h

    h = h_ref[...].astype(jnp.bfloat16)
    w2b = w2_ref[0].astype(jnp.bfloat16)
    lg = lax.dot_general(h, w2b, (((1,), (1,)), ((), ())),
                         preferred_element_type=jnp.float32) + b2_ref[0]
    out_ref[pl.ds(i, 1), :] = lg

    @pl.when(i == NV - 1)
    def _():
        allv = out_ref[...]
        m = jnp.max(allv)
        lse = m + jnp.log(jnp.sum(jnp.exp(allv - m)))
        out_ref[...] = allv - lse


_tc_fused = pl.pallas_call(
    _body,
    grid_spec=pltpu.PrefetchScalarGridSpec(
        num_scalar_prefetch=1,
        grid=(NV,),
        in_specs=[
            pl.BlockSpec(memory_space=pltpu.MemorySpace.HBM),
            pl.BlockSpec((H, D), lambda i, s: (0, 0)),
            pl.BlockSpec((1, H), lambda i, s: (0, 0)),
            pl.BlockSpec((1, R, H), lambda i, s: (i, 0, 0)),
            pl.BlockSpec((1, 1, R), lambda i, s: (i, 0, 0)),
        ],
        out_specs=[
            pl.BlockSpec((1, H), lambda i, s: (0, 0)),
            pl.BlockSpec((NV, R), lambda i, s: (0, 0)),
        ],
        scratch_shapes=[
            pltpu.VMEM((1, H), jnp.float32),
            pltpu.VMEM((L, D), jnp.float32),
            pltpu.SemaphoreType.DMA((NSEM,)),
        ],
    ),
    out_shape=[
        jax.ShapeDtypeStruct((1, H), jnp.float32),
        jax.ShapeDtypeStruct((NV, R), jnp.float32),
    ],
)


def kernel(_inputs, emb, W1, b1, W2, b2):
    idx = _inputs.astype(jnp.int32)
    proj, outr = _tc_fused(idx, emb, W1, b1.reshape(1, H),
                           W2.reshape(NV, R, H), b2.reshape(NV, 1, R))
    return (proj, outr.reshape(1, V))
